# baseline (speedup 1.0000x reference)
import functools

import jax
import jax.numpy as jnp
from jax.experimental import pallas as pl
from jax.experimental.pallas import tpu as pltpu

_MT = 256


def _round_up(x, m):
    return -(-x // m) * m


def _gru_tagger_kernel(ids_ref, emb_hbm, wih_ref, whh_ref, bih_ref, bhh_ref,
                       wout_ref, bout_ref, out_ref, embeds_ref, hs_ref, sem,
                       *, L, E, HP, T):
    KT = E // _MT
    ME = L // 4
    bf16 = jnp.bfloat16

    copies = []
    for t in range(L):
        c = pltpu.make_async_copy(
            emb_hbm.at[pl.ds(ids_ref[t], 1), :],
            embeds_ref.at[t],
            sem,
        )
        c.start()
        copies.append(c)
    for c in copies:
        c.wait()

    embeds = embeds_ref[...].reshape(L, E)
    xk = [embeds[:, k * _MT:(k + 1) * _MT].astype(bf16) for k in range(KT)]

    for g in range(3):
        mxu = g % 2
        addr = 2 * ME if g == 2 else 0
        for k in range(KT):
            pltpu.matmul_push_rhs(
                wih_ref[k * _MT:(k + 1) * _MT,
                        g * _MT:(g + 1) * _MT].astype(bf16),
                staging_register=k % 2, mxu_index=mxu)
        for k in range(KT):
            pltpu.matmul_acc_lhs(addr, xk[k], mxu_index=mxu,
                                 load_staged_rhs=k % 2)
    gi = [pltpu.matmul_pop(2 * ME if g == 2 else 0, (L, _MT), jnp.float32,
                           g % 2) for g in range(3)]

    whh_r_s = (whh_ref[:, 0 * _MT:1 * _MT] * 0.5).astype(bf16)
    whh_z_s = (whh_ref[:, 1 * _MT:2 * _MT] * 0.5).astype(bf16)
    pltpu.matmul_push_rhs(whh_ref[:, 2 * _MT:3 * _MT].astype(bf16),
                          staging_register=0, mxu_index=1)
    pltpu.matmul_push_rhs(wout_ref[...].astype(bf16),
                          staging_register=1, mxu_index=1)

    bhh_r = bhh_ref[:, 0 * _MT:1 * _MT]
    bhh_z = bhh_ref[:, 1 * _MT:2 * _MT]
    bhh_n = bhh_ref[:, 2 * _MT:3 * _MT]
    gi_r = 0.5 * (gi[0] + (bih_ref[:, 0 * _MT:1 * _MT] + bhh_r))
    gi_z = 0.5 * (gi[1] + (bih_ref[:, 1 * _MT:2 * _MT] + bhh_z))
    gi_n = gi[2] + bih_ref[:, 2 * _MT:3 * _MT]

    h = jnp.zeros((1, HP), jnp.float32)
    for t in range(L):
        if t == 0:
            arg_r = gi_r[0:1, :]
            arg_z = gi_z[0:1, :]
            a_n = bhh_n
            pre_n = gi_n[0:1, :]
        else:
            h16 = jnp.broadcast_to(h, (16, HP)).astype(bf16)
            pltpu.matmul_push_rhs(whh_r_s, staging_register=0, mxu_index=0)
            pltpu.matmul_push_rhs(whh_z_s, staging_register=1, mxu_index=0)
            pltpu.matmul_acc_lhs(0, h16, mxu_index=0, load_staged_rhs=0)
            pltpu.matmul_acc_lhs(4, h16, mxu_index=0, load_staged_rhs=1)
            pltpu.matmul_acc_lhs(0, h16, mxu_index=1,
                                 load_staged_rhs=0 if t == 1 else None)
            gh_r = pltpu.matmul_pop(0, (16, HP), jnp.float32, 0)[0:1, :]
            gh_z = pltpu.matmul_pop(4, (16, HP), jnp.float32, 0)[0:1, :]
            gh_n = pltpu.matmul_pop(0, (16, HP), jnp.float32, 1)[0:1, :]
            arg_r = gi_r[t:t + 1, :] + gh_r
            arg_z = gi_z[t:t + 1, :] + gh_z
            a_n = gh_n + bhh_n
            pre_n = gi_n[t:t + 1, :]
        t_r = jnp.tanh(arg_r)
        t_z = jnp.tanh(arg_z)
        half_a = 0.5 * a_n
        n = jnp.tanh((pre_n + half_a) + t_r * half_a)
        half_h = 0.5 * h
        zh = t_z * half_h + half_h
        omz = 0.5 - 0.5 * t_z
        h = omz * n + zh
        hs_ref[t:t + 1, :] = h

    pltpu.matmul_acc_lhs(2 * ME, hs_ref[...].astype(bf16), mxu_index=1,
                         load_staged_rhs=1)
    logits = (pltpu.matmul_pop(2 * ME, (L, _MT), jnp.float32, 1)[:, :T]
              + bout_ref[...])
    m = jnp.max(logits, axis=-1, keepdims=True)
    shifted = logits - m
    lse = jnp.log(jnp.sum(jnp.exp(shifted), axis=-1, keepdims=True))
    out_ref[...] = shifted - lse


def _pad_gate_cols(w, H, HP):
    if HP == H:
        return w
    lead = w.shape[:-1]
    w3 = w.reshape(lead + (3, H))
    w3 = jnp.pad(w3, [(0, 0)] * (len(lead) + 1) + [(0, HP - H)])
    return w3.reshape(lead + (3 * HP,))


def kernel(sentence, embedding, w_ih_t, w_hh_t, b_ih, b_hh, w_out_t, b_out):
    L = sentence.shape[0]
    E = embedding.shape[1]
    H = w_hh_t.shape[0]
    T = w_out_t.shape[1]
    HP = _round_up(H, 128)

    w_ih_p = _pad_gate_cols(w_ih_t, H, HP)
    w_hh_p = _pad_gate_cols(w_hh_t, H, HP)
    if HP != H:
        w_hh_p = jnp.pad(w_hh_p, ((0, HP - H), (0, 0)))
    b_ih_p = _pad_gate_cols(b_ih, H, HP)
    b_hh_p = _pad_gate_cols(b_hh, H, HP)
    w_out_p = jnp.pad(w_out_t, ((0, HP - H), (0, _MT - T)))

    ids = sentence.astype(jnp.int32)

    kernel_fn = functools.partial(_gru_tagger_kernel, L=L, E=E, HP=HP, T=T)
    return pl.pallas_call(
        kernel_fn,
        out_shape=jax.ShapeDtypeStruct((L, T), jnp.float32),
        in_specs=[
            pl.BlockSpec(memory_space=pltpu.SMEM),
            pl.BlockSpec(memory_space=pl.ANY),
            pl.BlockSpec(memory_space=pltpu.VMEM),
            pl.BlockSpec(memory_space=pltpu.VMEM),
            pl.BlockSpec(memory_space=pltpu.VMEM),
            pl.BlockSpec(memory_space=pltpu.VMEM),
            pl.BlockSpec(memory_space=pltpu.VMEM),
            pl.BlockSpec(memory_space=pltpu.VMEM),
        ],
        out_specs=pl.BlockSpec(memory_space=pltpu.VMEM),
        scratch_shapes=[
            pltpu.VMEM((L, 1, E), jnp.float32),
            pltpu.VMEM((L, HP), jnp.float32),
            pltpu.SemaphoreType.DMA,
        ],
        compiler_params=pltpu.CompilerParams(
            disable_bounds_checks=True,
        ),
    )(ids, embedding, w_ih_p, w_hh_p, b_ih_p, b_hh_p, w_out_p, b_out)


# speedup vs baseline: 1.1298x; 1.1298x over previous
import functools

import jax
import jax.numpy as jnp
from jax.experimental import pallas as pl
from jax.experimental.pallas import tpu as pltpu

_MT = 256


def _round_up(x, m):
    return -(-x // m) * m


def _gru_tagger_kernel(ids_ref, emb_hbm, wih_ref, whh_ref, bih_ref, bhh_ref,
                       wout_ref, bout_ref, out_ref, embeds_ref, hs_ref, sem,
                       *, L, E, HP, T):
    KT = E // _MT
    ME = L // 4
    bf16 = jnp.bfloat16

    copies = []
    for t in range(L):
        c = pltpu.make_async_copy(
            emb_hbm.at[pl.ds(ids_ref[t], 1), :],
            embeds_ref.at[t],
            sem,
        )
        c.start()
        copies.append(c)
    for c in copies:
        c.wait()

    embeds = embeds_ref[...].reshape(L, E)
    xk = [embeds[:, k * _MT:(k + 1) * _MT].astype(bf16) for k in range(KT)]

    for g in range(3):
        mxu = g % 2
        addr = 2 * ME if g == 2 else 0
        for k in range(KT):
            pltpu.matmul_push_rhs(
                wih_ref[k * _MT:(k + 1) * _MT,
                        g * _MT:(g + 1) * _MT].astype(bf16),
                staging_register=k % 2, mxu_index=mxu)
        for k in range(KT):
            pltpu.matmul_acc_lhs(addr, xk[k], mxu_index=mxu,
                                 load_staged_rhs=k % 2)
    gi = [pltpu.matmul_pop(2 * ME if g == 2 else 0, (L, _MT), jnp.float32,
                           g % 2) for g in range(3)]

    whh_r_s = (whh_ref[:, 0 * _MT:1 * _MT] * 0.5).astype(bf16)
    whh_z_s = (whh_ref[:, 1 * _MT:2 * _MT] * 0.5).astype(bf16)
    pltpu.matmul_push_rhs(whh_ref[:, 2 * _MT:3 * _MT].astype(bf16),
                          staging_register=0, mxu_index=1)
    if T < _MT:
        wout_tile = jnp.concatenate(
            [wout_ref[...], jnp.zeros((HP, _MT - T), jnp.float32)], axis=1)
    else:
        wout_tile = wout_ref[...]
    pltpu.matmul_push_rhs(wout_tile.astype(bf16),
                          staging_register=1, mxu_index=1)

    bhh_r = bhh_ref[:, 0 * _MT:1 * _MT]
    bhh_z = bhh_ref[:, 1 * _MT:2 * _MT]
    bhh_n = bhh_ref[:, 2 * _MT:3 * _MT]
    gi_r = 0.5 * (gi[0] + (bih_ref[:, 0 * _MT:1 * _MT] + bhh_r))
    gi_z = 0.5 * (gi[1] + (bih_ref[:, 1 * _MT:2 * _MT] + bhh_z))
    gi_n = gi[2] + bih_ref[:, 2 * _MT:3 * _MT]

    h = jnp.zeros((1, HP), jnp.float32)
    for t in range(L):
        if t == 0:
            arg_r = gi_r[0:1, :]
            arg_z = gi_z[0:1, :]
            a_n = bhh_n
            pre_n = gi_n[0:1, :]
        else:
            h16 = jnp.broadcast_to(h, (16, HP)).astype(bf16)
            pltpu.matmul_push_rhs(whh_r_s, staging_register=0, mxu_index=0)
            pltpu.matmul_push_rhs(whh_z_s, staging_register=1, mxu_index=0)
            pltpu.matmul_acc_lhs(0, h16, mxu_index=0, load_staged_rhs=0)
            pltpu.matmul_acc_lhs(4, h16, mxu_index=0, load_staged_rhs=1)
            pltpu.matmul_acc_lhs(0, h16, mxu_index=1,
                                 load_staged_rhs=0 if t == 1 else None)
            gh_r = pltpu.matmul_pop(0, (16, HP), jnp.float32, 0)[0:1, :]
            gh_z = pltpu.matmul_pop(4, (16, HP), jnp.float32, 0)[0:1, :]
            gh_n = pltpu.matmul_pop(0, (16, HP), jnp.float32, 1)[0:1, :]
            arg_r = gi_r[t:t + 1, :] + gh_r
            arg_z = gi_z[t:t + 1, :] + gh_z
            a_n = gh_n + bhh_n
            pre_n = gi_n[t:t + 1, :]
        t_r = jnp.tanh(arg_r)
        t_z = jnp.tanh(arg_z)
        half_a = 0.5 * a_n
        n = jnp.tanh((pre_n + half_a) + t_r * half_a)
        half_h = 0.5 * h
        zh = t_z * half_h + half_h
        omz = 0.5 - 0.5 * t_z
        h = omz * n + zh
        hs_ref[t:t + 1, :] = h

    pltpu.matmul_acc_lhs(2 * ME, hs_ref[...].astype(bf16), mxu_index=1,
                         load_staged_rhs=1)
    logits = (pltpu.matmul_pop(2 * ME, (L, _MT), jnp.float32, 1)[:, :T]
              + bout_ref[...])
    m = jnp.max(logits, axis=-1, keepdims=True)
    shifted = logits - m
    lse = jnp.log(jnp.sum(jnp.exp(shifted), axis=-1, keepdims=True))
    out_ref[...] = shifted - lse


def _pad_gate_cols(w, H, HP):
    if HP == H:
        return w
    lead = w.shape[:-1]
    w3 = w.reshape(lead + (3, H))
    w3 = jnp.pad(w3, [(0, 0)] * (len(lead) + 1) + [(0, HP - H)])
    return w3.reshape(lead + (3 * HP,))


def kernel(sentence, embedding, w_ih_t, w_hh_t, b_ih, b_hh, w_out_t, b_out):
    L = sentence.shape[0]
    E = embedding.shape[1]
    H = w_hh_t.shape[0]
    T = w_out_t.shape[1]
    HP = _round_up(H, 128)

    w_ih_p = _pad_gate_cols(w_ih_t, H, HP)
    w_hh_p = _pad_gate_cols(w_hh_t, H, HP)
    if HP != H:
        w_hh_p = jnp.pad(w_hh_p, ((0, HP - H), (0, 0)))
    b_ih_p = _pad_gate_cols(b_ih, H, HP)
    b_hh_p = _pad_gate_cols(b_hh, H, HP)
    w_out_p = jnp.pad(w_out_t, ((0, HP - H), (0, 0))) if HP != H else w_out_t

    ids = sentence.astype(jnp.int32)

    kernel_fn = functools.partial(_gru_tagger_kernel, L=L, E=E, HP=HP, T=T)
    return pl.pallas_call(
        kernel_fn,
        out_shape=jax.ShapeDtypeStruct((L, T), jnp.float32),
        in_specs=[
            pl.BlockSpec(memory_space=pltpu.SMEM),
            pl.BlockSpec(memory_space=pl.ANY),
            pl.BlockSpec(memory_space=pltpu.VMEM),
            pl.BlockSpec(memory_space=pltpu.VMEM),
            pl.BlockSpec(memory_space=pltpu.VMEM),
            pl.BlockSpec(memory_space=pltpu.VMEM),
            pl.BlockSpec(memory_space=pltpu.VMEM),
            pl.BlockSpec(memory_space=pltpu.VMEM),
        ],
        out_specs=pl.BlockSpec(memory_space=pltpu.VMEM),
        scratch_shapes=[
            pltpu.VMEM((L, 1, E), jnp.float32),
            pltpu.VMEM((L, HP), jnp.float32),
            pltpu.SemaphoreType.DMA,
        ],
        compiler_params=pltpu.CompilerParams(
            disable_bounds_checks=True,
        ),
    )(ids, embedding, w_ih_p, w_hh_p, b_ih_p, b_hh_p, w_out_p, b_out)


# speedup vs baseline: 1.1500x; 1.0178x over previous
import functools

import jax
import jax.numpy as jnp
from jax.experimental import pallas as pl
from jax.experimental.pallas import tpu as pltpu

_MT = 256


def _round_up(x, m):
    return -(-x // m) * m


def _gru_tagger_kernel(ids_ref, emb_hbm, wih_hbm, whh_hbm, bih_hbm, bhh_hbm,
                       wout_hbm, bout_hbm, out_ref, embeds_ref, wih_ref,
                       whh_ref, bih_ref, bhh_ref, wout_ref, bout_ref, hs_ref,
                       sem, wsem, *, L, E, HP, T):
    KT = E // _MT
    ME = L // 4
    bf16 = jnp.bfloat16

    copies = []
    for t in range(L):
        c = pltpu.make_async_copy(
            emb_hbm.at[pl.ds(ids_ref[t], 1), :],
            embeds_ref.at[t],
            sem,
        )
        c.start()
        copies.append(c)
    wcopies = [
        pltpu.make_async_copy(src, dst, wsem)
        for src, dst in [(wih_hbm, wih_ref), (whh_hbm, whh_ref),
                         (bih_hbm, bih_ref), (bhh_hbm, bhh_ref),
                         (wout_hbm, wout_ref), (bout_hbm, bout_ref)]
    ]
    for c in wcopies:
        c.start()
    for c in copies:
        c.wait()
    for c in wcopies:
        c.wait()

    embeds = embeds_ref[...].reshape(L, E)
    xk = [embeds[:, k * _MT:(k + 1) * _MT].astype(bf16) for k in range(KT)]

    for g in range(3):
        mxu = g % 2
        addr = 2 * ME if g == 2 else 0
        for k in range(KT):
            pltpu.matmul_push_rhs(
                wih_ref[k * _MT:(k + 1) * _MT,
                        g * _MT:(g + 1) * _MT].astype(bf16),
                staging_register=k % 2, mxu_index=mxu)
        for k in range(KT):
            pltpu.matmul_acc_lhs(addr, xk[k], mxu_index=mxu,
                                 load_staged_rhs=k % 2)
    gi = [pltpu.matmul_pop(2 * ME if g == 2 else 0, (L, _MT), jnp.float32,
                           g % 2) for g in range(3)]

    whh_r_s = (whh_ref[:, 0 * _MT:1 * _MT] * 0.5).astype(bf16)
    whh_z_s = (whh_ref[:, 1 * _MT:2 * _MT] * 0.5).astype(bf16)
    pltpu.matmul_push_rhs(whh_ref[:, 2 * _MT:3 * _MT].astype(bf16),
                          staging_register=0, mxu_index=1)
    if T < _MT:
        wout_tile = jnp.concatenate(
            [wout_ref[...], jnp.zeros((HP, _MT - T), jnp.float32)], axis=1)
    else:
        wout_tile = wout_ref[...]
    pltpu.matmul_push_rhs(wout_tile.astype(bf16),
                          staging_register=1, mxu_index=1)

    bhh_r = bhh_ref[:, 0 * _MT:1 * _MT]
    bhh_z = bhh_ref[:, 1 * _MT:2 * _MT]
    bhh_n = bhh_ref[:, 2 * _MT:3 * _MT]
    gi_r = 0.5 * (gi[0] + (bih_ref[:, 0 * _MT:1 * _MT] + bhh_r))
    gi_z = 0.5 * (gi[1] + (bih_ref[:, 1 * _MT:2 * _MT] + bhh_z))
    gi_n = gi[2] + bih_ref[:, 2 * _MT:3 * _MT]

    h = jnp.zeros((1, HP), jnp.float32)
    for t in range(L):
        if t == 0:
            arg_r = gi_r[0:1, :]
            arg_z = gi_z[0:1, :]
            a_n = bhh_n
            pre_n = gi_n[0:1, :]
        else:
            h16 = jnp.broadcast_to(h, (16, HP)).astype(bf16)
            pltpu.matmul_push_rhs(whh_r_s, staging_register=0, mxu_index=0)
            pltpu.matmul_push_rhs(whh_z_s, staging_register=1, mxu_index=0)
            pltpu.matmul_acc_lhs(0, h16, mxu_index=0, load_staged_rhs=0)
            pltpu.matmul_acc_lhs(4, h16, mxu_index=0, load_staged_rhs=1)
            pltpu.matmul_acc_lhs(0, h16, mxu_index=1,
                                 load_staged_rhs=0 if t == 1 else None)
            gh_r = pltpu.matmul_pop(0, (16, HP), jnp.float32, 0)[0:1, :]
            gh_z = pltpu.matmul_pop(4, (16, HP), jnp.float32, 0)[0:1, :]
            gh_n = pltpu.matmul_pop(0, (16, HP), jnp.float32, 1)[0:1, :]
            arg_r = gi_r[t:t + 1, :] + gh_r
            arg_z = gi_z[t:t + 1, :] + gh_z
            a_n = gh_n + bhh_n
            pre_n = gi_n[t:t + 1, :]
        t_r = jnp.tanh(arg_r)
        t_z = jnp.tanh(arg_z)
        half_a = 0.5 * a_n
        n = jnp.tanh((pre_n + half_a) + t_r * half_a)
        half_h = 0.5 * h
        zh = t_z * half_h + half_h
        omz = 0.5 - 0.5 * t_z
        h = omz * n + zh
        hs_ref[t:t + 1, :] = h

    pltpu.matmul_acc_lhs(2 * ME, hs_ref[...].astype(bf16), mxu_index=1,
                         load_staged_rhs=1)
    logits = (pltpu.matmul_pop(2 * ME, (L, _MT), jnp.float32, 1)[:, :T]
              + bout_ref[...])
    m = jnp.max(logits, axis=-1, keepdims=True)
    shifted = logits - m
    lse = jnp.log(jnp.sum(jnp.exp(shifted), axis=-1, keepdims=True))
    out_ref[...] = shifted - lse


def _pad_gate_cols(w, H, HP):
    if HP == H:
        return w
    lead = w.shape[:-1]
    w3 = w.reshape(lead + (3, H))
    w3 = jnp.pad(w3, [(0, 0)] * (len(lead) + 1) + [(0, HP - H)])
    return w3.reshape(lead + (3 * HP,))


def kernel(sentence, embedding, w_ih_t, w_hh_t, b_ih, b_hh, w_out_t, b_out):
    L = sentence.shape[0]
    E = embedding.shape[1]
    H = w_hh_t.shape[0]
    T = w_out_t.shape[1]
    HP = _round_up(H, 128)

    w_ih_p = _pad_gate_cols(w_ih_t, H, HP)
    w_hh_p = _pad_gate_cols(w_hh_t, H, HP)
    if HP != H:
        w_hh_p = jnp.pad(w_hh_p, ((0, HP - H), (0, 0)))
    b_ih_p = _pad_gate_cols(b_ih, H, HP)
    b_hh_p = _pad_gate_cols(b_hh, H, HP)
    w_out_p = jnp.pad(w_out_t, ((0, HP - H), (0, 0))) if HP != H else w_out_t

    ids = sentence.astype(jnp.int32)

    kernel_fn = functools.partial(_gru_tagger_kernel, L=L, E=E, HP=HP, T=T)
    return pl.pallas_call(
        kernel_fn,
        out_shape=jax.ShapeDtypeStruct((L, T), jnp.float32),
        in_specs=[
            pl.BlockSpec(memory_space=pltpu.SMEM),
            pl.BlockSpec(memory_space=pl.ANY),
            pl.BlockSpec(memory_space=pl.ANY),
            pl.BlockSpec(memory_space=pl.ANY),
            pl.BlockSpec(memory_space=pl.ANY),
            pl.BlockSpec(memory_space=pl.ANY),
            pl.BlockSpec(memory_space=pl.ANY),
            pl.BlockSpec(memory_space=pl.ANY),
        ],
        out_specs=pl.BlockSpec(memory_space=pltpu.VMEM),
        scratch_shapes=[
            pltpu.VMEM((L, 1, E), jnp.float32),
            pltpu.VMEM((E, 3 * HP), jnp.float32),
            pltpu.VMEM((HP, 3 * HP), jnp.float32),
            pltpu.VMEM((1, 3 * HP), jnp.float32),
            pltpu.VMEM((1, 3 * HP), jnp.float32),
            pltpu.VMEM((HP, T), jnp.float32),
            pltpu.VMEM((1, T), jnp.float32),
            pltpu.VMEM((L, HP), jnp.float32),
            pltpu.SemaphoreType.DMA,
            pltpu.SemaphoreType.DMA,
        ],
        compiler_params=pltpu.CompilerParams(
            disable_bounds_checks=True,
        ),
    )(ids, embedding, w_ih_p, w_hh_p, b_ih_p, b_hh_p, w_out_p, b_out)
